# baseline (speedup 1.0000x reference)
import functools

import jax
import jax.numpy as jnp
from jax import lax
from jax.experimental import pallas as pl
from jax.experimental.pallas import tpu as pltpu
from jax.experimental.pallas import tpu_sc as plsc

NC = 2
NS = 16
VL = 16


def _fill(ref, n, value):
    def body(i, _):
        ref[pl.ds(i * VL, VL)] = jnp.full((VL,), value, jnp.float32)
        return 0
    lax.fori_loop(0, n // VL, body, 0)


def _hist_body(np_, per_tile, chunk, edge_hbm, out_hbm, cnt_sh, dst_v, ones_v,
               zer_v):
    c = lax.axis_index("c")
    s = lax.axis_index("s")
    wid = c * NS + s
    slc = np_ // NS
    _fill(zer_v, slc, 0.0)
    _fill(ones_v, chunk, 1.0)
    pltpu.sync_copy(zer_v, cnt_sh.at[pl.ds(s * slc, slc)])
    plsc.subcore_barrier()
    base = wid * per_tile

    def chunk_body(k, _):
        pltpu.sync_copy(edge_hbm.at[1, pl.ds(base + k * chunk, chunk)], dst_v)
        pltpu.sync_copy(ones_v, cnt_sh.at[dst_v], add=True)
        return 0

    lax.fori_loop(0, per_tile // chunk, chunk_body, 0)
    plsc.subcore_barrier()
    pltpu.sync_copy(cnt_sh.at[pl.ds(s * slc, slc)], out_hbm.at[c, s])


def _gs_body(np_, per_tile, chunk, edge_hbm, t_hbm, out_hbm, acc_sh,
             t_v, src_v, dst_v, val_v, zer_v):
    c = lax.axis_index("c")
    s = lax.axis_index("s")
    wid = c * NS + s
    slc = np_ // NS
    _fill(zer_v, slc, 0.0)
    pltpu.sync_copy(zer_v, acc_sh.at[pl.ds(s * slc, slc)])
    pltpu.sync_copy(t_hbm, t_v)
    plsc.subcore_barrier()
    base = wid * per_tile

    def chunk_body(k, _):
        b = base + k * chunk
        pltpu.sync_copy(edge_hbm.at[0, pl.ds(b, chunk)], src_v)
        pltpu.sync_copy(edge_hbm.at[1, pl.ds(b, chunk)], dst_v)
        for j in range(chunk // VL):
            idx = src_v[pl.ds(j * VL, VL)]
            val_v[pl.ds(j * VL, VL)] = plsc.load_gather(t_v, [idx])
        pltpu.sync_copy(val_v, acc_sh.at[dst_v], add=True)
        return 0

    lax.fori_loop(0, per_tile // chunk, chunk_body, 0)
    plsc.subcore_barrier()
    pltpu.sync_copy(acc_sh.at[pl.ds(s * slc, slc)], out_hbm.at[c, s])


def _moments_body(x_ref, m2_ref, s1_ref):
    i = pl.program_id(0)
    xb = x_ref[...]

    @pl.when(i == 0)
    def _():
        m2_ref[...] = jnp.zeros_like(m2_ref)
        s1_ref[...] = jnp.zeros_like(s1_ref)

    m2_ref[...] += jax.lax.dot_general(
        xb, xb, (((0,), (0,)), ((), ())),
        preferred_element_type=jnp.float32,
        precision=lax.Precision.HIGHEST)
    s1_ref[...] += jnp.sum(xb, axis=0, keepdims=True)


def _encoder_body(n_true, x_ref, m2_ref, s1_ref, w1_ref, b1_ref, gamma_ref,
                  beta_ref, a_ref, w2_ref, b2_ref, wg_ref, wf_ref, y_ref):
    inv_n = 1.0 / n_true
    m = s1_ref[...] * inv_n
    e2 = m2_ref[...] * inv_n
    m0, m1 = m[0:1, 0:1], m[0:1, 1:2]
    v00 = e2[0:1, 0:1] - m0 * m0
    v01 = e2[0:1, 1:2] - m0 * m1
    v11 = e2[1:2, 1:2] - m1 * m1
    w1a = w1_ref[0:1, :]
    w1b = w1_ref[1:2, :]
    b1 = b1_ref[...]
    mu = m0 * w1a + m1 * w1b + b1
    var = v00 * w1a * w1a + 2.0 * v01 * w1a * w1b + v11 * w1b * w1b
    g = gamma_ref[...] * lax.rsqrt(var + 1e-5)
    w1aug = jnp.concatenate([g * w1a, g * w1b], axis=0)
    cv = g * (b1 - mu) + beta_ref[...]
    wgf = jax.lax.dot_general(
        wg_ref[...], wf_ref[...], (((1,), (0,)), ((), ())),
        preferred_element_type=jnp.float32,
        precision=lax.Precision.HIGHEST)
    w_eff = jax.lax.dot_general(
        w2_ref[...], wgf, (((1,), (0,)), ((), ())),
        preferred_element_type=jnp.float32,
        precision=lax.Precision.HIGHEST)
    y_const = jax.lax.dot_general(
        b2_ref[...], wgf, (((1,), (0,)), ((), ())),
        preferred_element_type=jnp.float32,
        precision=lax.Precision.HIGHEST)
    pre = jax.lax.dot_general(
        x_ref[...], w1aug, (((1,), (0,)), ((), ())),
        preferred_element_type=jnp.float32,
        precision=lax.Precision.HIGHEST) + cv
    enc = jnp.where(pre >= 0, pre, a_ref[...] * pre)
    y_ref[...] = jax.lax.dot_general(
        enc, w_eff, (((1,), (0,)), ((), ())),
        preferred_element_type=jnp.float32,
        precision=lax.Precision.HIGHEST) + y_const


def _t_body(c0_ref, c1_ref, y_ref, t_ref, dinv_ref):
    deg = c0_ref[...] + c1_ref[...] + 1.0
    dinv = lax.rsqrt(deg)
    t_ref[...] = dinv * y_ref[...]
    dinv_ref[...] = dinv


def _combine_body(a0_ref, a1_ref, t_ref, dinv_ref, bg_ref, wf_ref, bf_ref,
                  out_ref):
    cst = (jax.lax.dot_general(
        bg_ref[...], wf_ref[...], (((1,), (0,)), ((), ())),
        preferred_element_type=jnp.float32,
        precision=lax.Precision.HIGHEST) + bf_ref[...])
    out_ref[...] = (dinv_ref[...]
                    * (a0_ref[...] + a1_ref[...] + t_ref[...]) + cst)


def kernel(x, edge_index, W1, b1, gamma, beta, prelu_a, W2, b2, Wg, bg, Wf, bf):
    N = x.shape[0]
    E = edge_index.shape[1]
    np_ = ((N + 127) // 128) * 128
    rows = np_ // 128
    slc = np_ // NS
    per_tile = E // (NC * NS)
    chunk = 2000
    bn = 4352
    while np_ % bn:
        bn -= 128
    nb = np_ // bn
    f32 = jnp.float32

    xp = jnp.pad(x, ((0, np_ - N), (0, 0)))

    mesh = plsc.VectorSubcoreMesh(core_axis_name="c", subcore_axis_name="s")
    sc_params = pltpu.CompilerParams(use_tc_tiling_on_sc=False,
                                     needs_layout_passes=False)
    cnt = pl.kernel(
        functools.partial(_hist_body, np_, per_tile, chunk),
        out_type=jax.ShapeDtypeStruct((NC, NS, slc), f32),
        mesh=mesh,
        compiler_params=sc_params,
        scratch_types=[
            pltpu.VMEM_SHARED((np_,), f32),
            pltpu.VMEM((chunk,), jnp.int32),
            pltpu.VMEM((chunk,), f32),
            pltpu.VMEM((slc,), f32),
        ],
    )(edge_index)
    cnt_r = cnt.reshape(NC, rows, 128)

    m2, s1 = pl.pallas_call(
        _moments_body,
        grid=(nb,),
        in_specs=[pl.BlockSpec((bn, 2), lambda i: (i, 0))],
        out_specs=[pl.BlockSpec((2, 2), lambda i: (0, 0)),
                   pl.BlockSpec((1, 2), lambda i: (0, 0))],
        out_shape=[jax.ShapeDtypeStruct((2, 2), f32),
                   jax.ShapeDtypeStruct((1, 2), f32)],
    )(xp)

    full = pl.BlockSpec(memory_space=pltpu.VMEM)
    y = pl.pallas_call(
        functools.partial(_encoder_body, float(N)),
        grid=(nb,),
        in_specs=[pl.BlockSpec((bn, 2), lambda i: (i, 0))] + [full] * 11,
        out_specs=pl.BlockSpec((bn, 1), lambda i: (i, 0)),
        out_shape=jax.ShapeDtypeStruct((np_, 1), f32),
    )(xp, m2, s1, W1, b1.reshape(1, 32), gamma.reshape(1, 32),
      beta.reshape(1, 32), prelu_a.reshape(1, 1), W2, b2.reshape(1, 32),
      Wg, Wf, )
    y_r = y.reshape(rows, 128)

    t_r, dinv_r = pl.pallas_call(
        _t_body,
        out_shape=[jax.ShapeDtypeStruct((rows, 128), f32),
                   jax.ShapeDtypeStruct((rows, 128), f32)],
    )(cnt_r[0], cnt_r[1], y_r)

    acc = pl.kernel(
        functools.partial(_gs_body, np_, per_tile, chunk),
        out_type=jax.ShapeDtypeStruct((NC, NS, slc), f32),
        mesh=mesh,
        compiler_params=sc_params,
        scratch_types=[
            pltpu.VMEM_SHARED((np_,), f32),
            pltpu.VMEM((np_,), f32),
            pltpu.VMEM((chunk,), jnp.int32),
            pltpu.VMEM((chunk,), jnp.int32),
            pltpu.VMEM((chunk,), f32),
            pltpu.VMEM((slc,), f32),
        ],
    )(edge_index, t_r.reshape(np_))
    acc_r = acc.reshape(NC, rows, 128)

    scores_r = pl.pallas_call(
        _combine_body,
        out_shape=jax.ShapeDtypeStruct((rows, 128), f32),
    )(acc_r[0], acc_r[1], t_r, dinv_r, bg.reshape(1, 32), Wf,
      bf.reshape(1, 1))
    return scores_r.reshape(np_)[:N]


# speedup vs baseline: 2.3919x; 2.3919x over previous
import functools

import jax
import jax.numpy as jnp
from jax import lax
from jax.experimental import pallas as pl
from jax.experimental.pallas import tpu as pltpu
from jax.experimental.pallas import tpu_sc as plsc

NC = 2
NS = 16
VL = 16


def _bf16r(v):
    return v.astype(jnp.bfloat16).astype(jnp.float32)


def _dot00(a, b):
    return jax.lax.dot_general(
        a, b, (((0,), (0,)), ((), ())),
        preferred_element_type=jnp.float32)


def _fill(ref, n, value):
    def body(i, _):
        ref[pl.ds(i * VL, VL)] = jnp.full((VL,), value, jnp.float32)
        return 0
    lax.fori_loop(0, n // VL, body, 0)


def _hist_body(np_, per_tile, chunk, edge_hbm, out_hbm, cnt_sh, dst_v, ones_v,
               zer_v):
    c = lax.axis_index("c")
    s = lax.axis_index("s")
    wid = c * NS + s
    slc = np_ // NS
    _fill(zer_v, slc, 0.0)
    _fill(ones_v, chunk, 1.0)
    pltpu.sync_copy(zer_v, cnt_sh.at[pl.ds(s * slc, slc)])
    plsc.subcore_barrier()
    base = wid * per_tile

    def chunk_body(k, _):
        pltpu.sync_copy(edge_hbm.at[1, pl.ds(base + k * chunk, chunk)], dst_v)
        pltpu.sync_copy(ones_v, cnt_sh.at[dst_v], add=True)
        return 0

    lax.fori_loop(0, per_tile // chunk, chunk_body, 0)
    plsc.subcore_barrier()
    pltpu.sync_copy(cnt_sh.at[pl.ds(s * slc, slc)], out_hbm.at[c, s])


def _gs_body(np_, per_tile, chunk, edge_hbm, t_hbm, out_hbm, acc_sh,
             t_v, src_v, dst_v, val_v, zer_v):
    c = lax.axis_index("c")
    s = lax.axis_index("s")
    wid = c * NS + s
    slc = np_ // NS
    _fill(zer_v, slc, 0.0)
    pltpu.sync_copy(zer_v, acc_sh.at[pl.ds(s * slc, slc)])
    pltpu.sync_copy(t_hbm, t_v)
    plsc.subcore_barrier()
    base = wid * per_tile

    def chunk_body(k, _):
        b = base + k * chunk
        pltpu.sync_copy(edge_hbm.at[0, pl.ds(b, chunk)], src_v)
        pltpu.sync_copy(edge_hbm.at[1, pl.ds(b, chunk)], dst_v)
        for j in range(chunk // VL):
            idx = src_v[pl.ds(j * VL, VL)]
            val_v[pl.ds(j * VL, VL)] = plsc.load_gather(t_v, [idx])
        pltpu.sync_copy(val_v, acc_sh.at[dst_v], add=True)
        return 0

    lax.fori_loop(0, per_tile // chunk, chunk_body, 0)
    plsc.subcore_barrier()
    pltpu.sync_copy(acc_sh.at[pl.ds(s * slc, slc)], out_hbm.at[c, s])


def _moments_body(x0_ref, x1_ref, out_ref):
    x0 = _bf16r(x0_ref[...])
    x1 = _bf16r(x1_ref[...])
    out_ref[0:1, :] = jnp.sum(x0, axis=0, keepdims=True)
    out_ref[1:2, :] = jnp.sum(x1, axis=0, keepdims=True)
    out_ref[2:3, :] = jnp.sum(x0 * x0, axis=0, keepdims=True)
    out_ref[3:4, :] = jnp.sum(x1 * x1, axis=0, keepdims=True)
    out_ref[4:5, :] = jnp.sum(x0 * x1, axis=0, keepdims=True)
    out_ref[5:8, :] = jnp.zeros((3, 128), jnp.float32)


def _encoder_body(rblk, x0_ref, x1_ref, a_ref, acol_ref, bcol_ref, ccol_ref,
                  w2b_ref, b2col_ref, wgf_ref, y_ref):
    a = a_ref[0, 0]
    acol = acol_ref[...]
    bcol = bcol_ref[...]
    ccol = ccol_ref[...]
    w2b = w2b_ref[...]
    b2col = b2col_ref[...]
    wgf = wgf_ref[...]
    for r in range(rblk):
        xb0 = _bf16r(x0_ref[r:r + 1, :])
        xb1 = _bf16r(x1_ref[r:r + 1, :])
        pre = acol * xb0 + bcol * xb1 + ccol
        enc = jnp.where(pre >= 0, pre, a * pre)
        h2 = _dot00(w2b, _bf16r(enc)) + b2col
        h2b = _bf16r(h2)
        y_ref[r:r + 1, :] = jnp.sum(wgf * h2b, axis=0, keepdims=True)
    del y_ref


def _t_body(c0_ref, c1_ref, y_ref, t_ref, dinv_ref):
    deg = c0_ref[...] + c1_ref[...] + 1.0
    dinv = lax.rsqrt(deg)
    t_ref[...] = dinv * y_ref[...]
    dinv_ref[...] = dinv


def _combine_body(p_ref, a0_ref, a1_ref, t_ref, dinv_ref, out_ref):
    cst = p_ref[0, 0]
    out_ref[...] = (dinv_ref[...]
                    * (a0_ref[...] + a1_ref[...] + t_ref[...]) + cst)


def kernel(x, edge_index, W1, b1, gamma, beta, prelu_a, W2, b2, Wg, bg, Wf, bf):
    N = x.shape[0]
    E = edge_index.shape[1]
    np_ = ((N + 1023) // 1024) * 1024
    rows = np_ // 128
    slc = np_ // NS
    per_tile = E // (NC * NS)
    chunk = 2000
    rblk = 56
    while rows % rblk:
        rblk -= 8
    f32 = jnp.float32

    pad = np_ - N
    x0p = jnp.pad(x[:, 0], (0, pad)).reshape(rows, 128)
    x1p = jnp.pad(x[:, 1], (0, pad)).reshape(rows, 128)

    mesh = plsc.VectorSubcoreMesh(core_axis_name="c", subcore_axis_name="s")
    sc_params = pltpu.CompilerParams(use_tc_tiling_on_sc=False,
                                     needs_layout_passes=False)
    cnt = pl.kernel(
        functools.partial(_hist_body, np_, per_tile, chunk),
        out_type=jax.ShapeDtypeStruct((NC, NS, slc), f32),
        mesh=mesh,
        compiler_params=sc_params,
        scratch_types=[
            pltpu.VMEM_SHARED((np_,), f32),
            pltpu.VMEM((chunk,), jnp.int32),
            pltpu.VMEM((chunk,), f32),
            pltpu.VMEM((slc,), f32),
        ],
    )(edge_index)
    cnt_r = cnt.reshape(NC, rows, 128)

    mom = pl.pallas_call(
        _moments_body,
        out_shape=jax.ShapeDtypeStruct((8, 128), f32),
    )(x0p, x1p)
    sums = jnp.sum(mom, axis=1)
    n_f = jnp.float32(N)
    m0, m1 = sums[0] / n_f, sums[1] / n_f
    e00, e11, e01 = sums[2] / n_f, sums[3] / n_f, sums[4] / n_f
    v00 = e00 - m0 * m0
    v01 = e01 - m0 * m1
    v11 = e11 - m1 * m1

    w1b = _bf16r(W1)
    mu = m0 * w1b[0] + m1 * w1b[1] + b1
    var = v00 * w1b[0] ** 2 + 2.0 * v01 * w1b[0] * w1b[1] + v11 * w1b[1] ** 2
    g = gamma * lax.rsqrt(var + 1e-5)
    acol = (g * w1b[0]).reshape(32, 1)
    bcol = (g * w1b[1]).reshape(32, 1)
    ccol = (g * (b1 - mu) + beta).reshape(32, 1)
    wgf = jnp.dot(Wg, Wf)
    cst = jnp.dot(bg, Wf)[0] + bf[0]
    dparams = jnp.reshape(cst, (1, 1))

    full = pl.BlockSpec(memory_space=pltpu.VMEM)
    y_r = pl.pallas_call(
        functools.partial(_encoder_body, rblk),
        grid=(rows // rblk,),
        in_specs=[pl.BlockSpec((rblk, 128), lambda i: (i, 0)),
                  pl.BlockSpec((rblk, 128), lambda i: (i, 0)),
                  pl.BlockSpec(memory_space=pltpu.SMEM),
                  full, full, full, full, full, full],
        out_specs=pl.BlockSpec((rblk, 128), lambda i: (i, 0)),
        out_shape=jax.ShapeDtypeStruct((rows, 128), f32),
    )(x0p, x1p, prelu_a.reshape(1, 1), acol, bcol, ccol, _bf16r(W2),
      b2.reshape(32, 1), wgf)

    t_r, dinv_r = pl.pallas_call(
        _t_body,
        out_shape=[jax.ShapeDtypeStruct((rows, 128), f32),
                   jax.ShapeDtypeStruct((rows, 128), f32)],
    )(cnt_r[0], cnt_r[1], y_r)

    acc = pl.kernel(
        functools.partial(_gs_body, np_, per_tile, chunk),
        out_type=jax.ShapeDtypeStruct((NC, NS, slc), f32),
        mesh=mesh,
        compiler_params=sc_params,
        scratch_types=[
            pltpu.VMEM_SHARED((np_,), f32),
            pltpu.VMEM((np_,), f32),
            pltpu.VMEM((chunk,), jnp.int32),
            pltpu.VMEM((chunk,), jnp.int32),
            pltpu.VMEM((chunk,), f32),
            pltpu.VMEM((slc,), f32),
        ],
    )(edge_index, t_r.reshape(np_))
    acc_r = acc.reshape(NC, rows, 128)

    scores_r = pl.pallas_call(
        _combine_body,
        in_specs=[pl.BlockSpec(memory_space=pltpu.SMEM),
                  full, full, full, full],
        out_shape=jax.ShapeDtypeStruct((rows, 128), f32),
    )(dparams, acc_r[0], acc_r[1], t_r, dinv_r)
    return scores_r.reshape(np_)[:N]


# speedup vs baseline: 2.5986x; 1.0865x over previous
import functools

import jax
import jax.numpy as jnp
from jax import lax
from jax.experimental import pallas as pl
from jax.experimental.pallas import tpu as pltpu
from jax.experimental.pallas import tpu_sc as plsc

NC = 2
NS = 16
VL = 16


def _bf16r(v):
    return v.astype(jnp.bfloat16).astype(jnp.float32)


def _dot00(a, b):
    return jax.lax.dot_general(
        a, b, (((0,), (0,)), ((), ())),
        preferred_element_type=jnp.float32)


def _fill(ref, n, value):
    def body(i, _):
        ref[pl.ds(i * VL, VL)] = jnp.full((VL,), value, jnp.float32)
        return 0
    lax.fori_loop(0, n // VL, body, 0)


def _hist_body(np_, per_tile, chunk, edge_hbm, out_hbm, cnt_sh, dst_v, ones_v,
               zer_v):
    c = lax.axis_index("c")
    s = lax.axis_index("s")
    wid = c * NS + s
    slc = np_ // NS
    _fill(zer_v, slc, 0.0)
    _fill(ones_v, chunk, 1.0)
    pltpu.sync_copy(zer_v, cnt_sh.at[pl.ds(s * slc, slc)])
    plsc.subcore_barrier()
    base = wid * per_tile

    def chunk_body(k, _):
        pltpu.sync_copy(edge_hbm.at[1, pl.ds(base + k * chunk, chunk)], dst_v)
        pltpu.sync_copy(ones_v, cnt_sh.at[dst_v], add=True)
        return 0

    lax.fori_loop(0, per_tile // chunk, chunk_body, 0)
    plsc.subcore_barrier()
    pltpu.sync_copy(cnt_sh.at[pl.ds(s * slc, slc)], out_hbm.at[c, s])


def _gs_body(np_, per_tile, chunk, edge_hbm, t_hbm, out_hbm, acc_sh,
             t_v, src_a, dst_a, val_a, src_b, dst_b, val_b, zer_v,
             sem_a, sem_b):
    c = lax.axis_index("c")
    s = lax.axis_index("s")
    wid = c * NS + s
    slc = np_ // NS
    _fill(zer_v, slc, 0.0)
    pltpu.sync_copy(zer_v, acc_sh.at[pl.ds(s * slc, slc)])
    pltpu.sync_copy(t_hbm, t_v)
    plsc.subcore_barrier()
    base = wid * per_tile
    nchunks = per_tile // chunk

    def load(b, src_v, dst_v):
        pltpu.sync_copy(edge_hbm.at[0, pl.ds(b, chunk)], src_v)
        pltpu.sync_copy(edge_hbm.at[1, pl.ds(b, chunk)], dst_v)

    def gather(src_v, val_v):
        for j in range(chunk // VL):
            idx = src_v[pl.ds(j * VL, VL)]
            val_v[pl.ds(j * VL, VL)] = plsc.load_gather(t_v, [idx])

    def scat(val_v, dst_v, sem):
        return pltpu.make_async_copy(val_v, acc_sh.at[dst_v], sem)

    load(base, src_a, dst_a)
    gather(src_a, val_a)
    scat(val_a, dst_a, sem_a).start(add=True)

    def pair_body(m, _):
        b = base + (2 * m + 1) * chunk
        load(b, src_b, dst_b)
        gather(src_b, val_b)
        scat(val_a, dst_a, sem_a).wait()
        scat(val_b, dst_b, sem_b).start(add=True)
        b2 = base + (2 * m + 2) * chunk
        load(b2, src_a, dst_a)
        gather(src_a, val_a)
        scat(val_b, dst_b, sem_b).wait()
        scat(val_a, dst_a, sem_a).start(add=True)
        return 0

    lax.fori_loop(0, (nchunks - 1) // 2, pair_body, 0)
    scat(val_a, dst_a, sem_a).wait()
    plsc.subcore_barrier()
    pltpu.sync_copy(acc_sh.at[pl.ds(s * slc, slc)], out_hbm.at[c, s])


def _moments_body(x0_ref, x1_ref, out_ref):
    x0 = _bf16r(x0_ref[...])
    x1 = _bf16r(x1_ref[...])
    out_ref[0:1, :] = jnp.sum(x0, axis=0, keepdims=True)
    out_ref[1:2, :] = jnp.sum(x1, axis=0, keepdims=True)
    out_ref[2:3, :] = jnp.sum(x0 * x0, axis=0, keepdims=True)
    out_ref[3:4, :] = jnp.sum(x1 * x1, axis=0, keepdims=True)
    out_ref[4:5, :] = jnp.sum(x0 * x1, axis=0, keepdims=True)
    out_ref[5:8, :] = jnp.zeros((3, 128), jnp.float32)


def _encoder_body(rblk, x0_ref, x1_ref, a_ref, acol_ref, bcol_ref, ccol_ref,
                  w2b_ref, b2col_ref, wgf_ref, y_ref):
    a = a_ref[0, 0]
    acol = acol_ref[...]
    bcol = bcol_ref[...]
    ccol = ccol_ref[...]
    w2b = w2b_ref[...]
    b2col = b2col_ref[...]
    wgf = wgf_ref[...]
    for r in range(rblk):
        xb0 = _bf16r(x0_ref[r:r + 1, :])
        xb1 = _bf16r(x1_ref[r:r + 1, :])
        pre = acol * xb0 + bcol * xb1 + ccol
        enc = jnp.where(pre >= 0, pre, a * pre)
        h2 = _dot00(w2b, _bf16r(enc)) + b2col
        h2b = _bf16r(h2)
        y_ref[r:r + 1, :] = jnp.sum(wgf * h2b, axis=0, keepdims=True)
    del y_ref


def _t_body(c0_ref, c1_ref, y_ref, t_ref, dinv_ref):
    deg = c0_ref[...] + c1_ref[...] + 1.0
    dinv = lax.rsqrt(deg)
    t_ref[...] = dinv * y_ref[...]
    dinv_ref[...] = dinv


def _combine_body(p_ref, a0_ref, a1_ref, t_ref, dinv_ref, out_ref):
    cst = p_ref[0, 0]
    out_ref[...] = (dinv_ref[...]
                    * (a0_ref[...] + a1_ref[...] + t_ref[...]) + cst)


def kernel(x, edge_index, W1, b1, gamma, beta, prelu_a, W2, b2, Wg, bg, Wf, bf):
    N = x.shape[0]
    E = edge_index.shape[1]
    np_ = ((N + 1023) // 1024) * 1024
    rows = np_ // 128
    slc = np_ // NS
    per_tile = E // (NC * NS)
    chunk = 2000
    rblk = 56
    while rows % rblk:
        rblk -= 8
    f32 = jnp.float32

    pad = np_ - N
    x0p = jnp.pad(x[:, 0], (0, pad)).reshape(rows, 128)
    x1p = jnp.pad(x[:, 1], (0, pad)).reshape(rows, 128)

    mesh = plsc.VectorSubcoreMesh(core_axis_name="c", subcore_axis_name="s")
    sc_params = pltpu.CompilerParams(use_tc_tiling_on_sc=False,
                                     needs_layout_passes=False)
    cnt = pl.kernel(
        functools.partial(_hist_body, np_, per_tile, chunk),
        out_type=jax.ShapeDtypeStruct((NC, NS, slc), f32),
        mesh=mesh,
        compiler_params=sc_params,
        scratch_types=[
            pltpu.VMEM_SHARED((np_,), f32),
            pltpu.VMEM((chunk,), jnp.int32),
            pltpu.VMEM((chunk,), f32),
            pltpu.VMEM((slc,), f32),
        ],
    )(edge_index)
    cnt_r = cnt.reshape(NC, rows, 128)

    mom = pl.pallas_call(
        _moments_body,
        out_shape=jax.ShapeDtypeStruct((8, 128), f32),
    )(x0p, x1p)
    sums = jnp.sum(mom, axis=1)
    n_f = jnp.float32(N)
    m0, m1 = sums[0] / n_f, sums[1] / n_f
    e00, e11, e01 = sums[2] / n_f, sums[3] / n_f, sums[4] / n_f
    v00 = e00 - m0 * m0
    v01 = e01 - m0 * m1
    v11 = e11 - m1 * m1

    w1b = _bf16r(W1)
    mu = m0 * w1b[0] + m1 * w1b[1] + b1
    var = v00 * w1b[0] ** 2 + 2.0 * v01 * w1b[0] * w1b[1] + v11 * w1b[1] ** 2
    g = gamma * lax.rsqrt(var + 1e-5)
    acol = (g * w1b[0]).reshape(32, 1)
    bcol = (g * w1b[1]).reshape(32, 1)
    ccol = (g * (b1 - mu) + beta).reshape(32, 1)
    wgf = jnp.dot(Wg, Wf)
    cst = jnp.dot(bg, Wf)[0] + bf[0]
    dparams = jnp.reshape(cst, (1, 1))

    full = pl.BlockSpec(memory_space=pltpu.VMEM)
    y_r = pl.pallas_call(
        functools.partial(_encoder_body, rblk),
        grid=(rows // rblk,),
        in_specs=[pl.BlockSpec((rblk, 128), lambda i: (i, 0)),
                  pl.BlockSpec((rblk, 128), lambda i: (i, 0)),
                  pl.BlockSpec(memory_space=pltpu.SMEM),
                  full, full, full, full, full, full],
        out_specs=pl.BlockSpec((rblk, 128), lambda i: (i, 0)),
        out_shape=jax.ShapeDtypeStruct((rows, 128), f32),
    )(x0p, x1p, prelu_a.reshape(1, 1), acol, bcol, ccol, _bf16r(W2),
      b2.reshape(32, 1), wgf)

    t_r, dinv_r = pl.pallas_call(
        _t_body,
        out_shape=[jax.ShapeDtypeStruct((rows, 128), f32),
                   jax.ShapeDtypeStruct((rows, 128), f32)],
    )(cnt_r[0], cnt_r[1], y_r)

    acc = pl.kernel(
        functools.partial(_gs_body, np_, per_tile, chunk),
        out_type=jax.ShapeDtypeStruct((NC, NS, slc), f32),
        mesh=mesh,
        compiler_params=sc_params,
        scratch_types=[
            pltpu.VMEM_SHARED((np_,), f32),
            pltpu.VMEM((np_,), f32),
            pltpu.VMEM((chunk,), jnp.int32),
            pltpu.VMEM((chunk,), jnp.int32),
            pltpu.VMEM((chunk,), f32),
            pltpu.VMEM((chunk,), jnp.int32),
            pltpu.VMEM((chunk,), jnp.int32),
            pltpu.VMEM((chunk,), f32),
            pltpu.VMEM((slc,), f32),
            pltpu.SemaphoreType.DMA,
            pltpu.SemaphoreType.DMA,
        ],
    )(edge_index, t_r.reshape(np_))
    acc_r = acc.reshape(NC, rows, 128)

    scores_r = pl.pallas_call(
        _combine_body,
        in_specs=[pl.BlockSpec(memory_space=pltpu.SMEM),
                  full, full, full, full],
        out_shape=jax.ShapeDtypeStruct((rows, 128), f32),
    )(dparams, acc_r[0], acc_r[1], t_r, dinv_r)
    return scores_r.reshape(np_)[:N]


# speedup vs baseline: 2.7255x; 1.0488x over previous
import functools

import jax
import jax.numpy as jnp
from jax import lax
from jax.experimental import pallas as pl
from jax.experimental.pallas import tpu as pltpu
from jax.experimental.pallas import tpu_sc as plsc

NC = 2
NS = 16
VL = 16


def _bf16r(v):
    return v.astype(jnp.bfloat16).astype(jnp.float32)


def _dot00(a, b):
    return jax.lax.dot_general(
        a, b, (((0,), (0,)), ((), ())),
        preferred_element_type=jnp.float32)


def _fill(ref, n, value):
    def body(i, _):
        ref[pl.ds(i * VL, VL)] = jnp.full((VL,), value, jnp.float32)
        return 0
    lax.fori_loop(0, n // VL, body, 0)


def _hist_body(np_, per_tile, chunk, edge_hbm, out_hbm, cnt_sh, dst_a, dst_b,
               ones_v, zer_v, sem_a, sem_b):
    c = lax.axis_index("c")
    s = lax.axis_index("s")
    wid = c * NS + s
    slc = np_ // NS
    _fill(zer_v, slc, 0.0)
    _fill(ones_v, chunk, 1.0)
    pltpu.sync_copy(zer_v, cnt_sh.at[pl.ds(s * slc, slc)])
    plsc.subcore_barrier()
    base = wid * per_tile

    def scat(dst_v, sem):
        return pltpu.make_async_copy(ones_v, cnt_sh.at[dst_v], sem)

    pltpu.sync_copy(edge_hbm.at[1, pl.ds(base, chunk)], dst_a)
    scat(dst_a, sem_a).start(add=True)

    def pair_body(m, _):
        b = base + (2 * m + 1) * chunk
        pltpu.sync_copy(edge_hbm.at[1, pl.ds(b, chunk)], dst_b)
        scat(dst_a, sem_a).wait()
        scat(dst_b, sem_b).start(add=True)
        b2 = base + (2 * m + 2) * chunk
        pltpu.sync_copy(edge_hbm.at[1, pl.ds(b2, chunk)], dst_a)
        scat(dst_b, sem_b).wait()
        scat(dst_a, sem_a).start(add=True)
        return 0

    lax.fori_loop(0, (per_tile // chunk - 1) // 2, pair_body, 0)
    scat(dst_a, sem_a).wait()
    plsc.subcore_barrier()
    pltpu.sync_copy(cnt_sh.at[pl.ds(s * slc, slc)], out_hbm.at[c, s])


def _gs_body(np_, per_tile, chunk, edge_hbm, t_hbm, out_hbm, acc_sh,
             t_v, src_a, dst_a, val_a, src_b, dst_b, val_b, zer_v,
             sem_a, sem_b):
    c = lax.axis_index("c")
    s = lax.axis_index("s")
    wid = c * NS + s
    slc = np_ // NS
    _fill(zer_v, slc, 0.0)
    pltpu.sync_copy(zer_v, acc_sh.at[pl.ds(s * slc, slc)])
    pltpu.sync_copy(t_hbm, t_v)
    plsc.subcore_barrier()
    base = wid * per_tile
    nchunks = per_tile // chunk

    def load(b, src_v, dst_v):
        pltpu.sync_copy(edge_hbm.at[0, pl.ds(b, chunk)], src_v)
        pltpu.sync_copy(edge_hbm.at[1, pl.ds(b, chunk)], dst_v)

    def gather(src_v, val_v):
        for j in range(chunk // VL):
            idx = src_v[pl.ds(j * VL, VL)]
            val_v[pl.ds(j * VL, VL)] = plsc.load_gather(t_v, [idx])

    def scat(val_v, dst_v, sem):
        return pltpu.make_async_copy(val_v, acc_sh.at[dst_v], sem)

    load(base, src_a, dst_a)
    gather(src_a, val_a)
    scat(val_a, dst_a, sem_a).start(add=True)

    def pair_body(m, _):
        b = base + (2 * m + 1) * chunk
        load(b, src_b, dst_b)
        gather(src_b, val_b)
        scat(val_a, dst_a, sem_a).wait()
        scat(val_b, dst_b, sem_b).start(add=True)
        b2 = base + (2 * m + 2) * chunk
        load(b2, src_a, dst_a)
        gather(src_a, val_a)
        scat(val_b, dst_b, sem_b).wait()
        scat(val_a, dst_a, sem_a).start(add=True)
        return 0

    lax.fori_loop(0, (nchunks - 1) // 2, pair_body, 0)
    scat(val_a, dst_a, sem_a).wait()
    plsc.subcore_barrier()
    pltpu.sync_copy(acc_sh.at[pl.ds(s * slc, slc)], out_hbm.at[c, s])


def _moments_body(x0_ref, x1_ref, out_ref):
    x0 = _bf16r(x0_ref[...])
    x1 = _bf16r(x1_ref[...])
    out_ref[0:1, :] = jnp.sum(x0, axis=0, keepdims=True)
    out_ref[1:2, :] = jnp.sum(x1, axis=0, keepdims=True)
    out_ref[2:3, :] = jnp.sum(x0 * x0, axis=0, keepdims=True)
    out_ref[3:4, :] = jnp.sum(x1 * x1, axis=0, keepdims=True)
    out_ref[4:5, :] = jnp.sum(x0 * x1, axis=0, keepdims=True)
    out_ref[5:8, :] = jnp.zeros((3, 128), jnp.float32)


def _encoder_body(rblk, x0_ref, x1_ref, a_ref, acol_ref, bcol_ref, ccol_ref,
                  w2b_ref, b2col_ref, wgf_ref, y_ref):
    a = a_ref[0, 0]
    acol = acol_ref[...]
    bcol = bcol_ref[...]
    ccol = ccol_ref[...]
    w2b = w2b_ref[...]
    b2col = b2col_ref[...]
    wgf = wgf_ref[...]
    for r in range(rblk):
        xb0 = _bf16r(x0_ref[r:r + 1, :])
        xb1 = _bf16r(x1_ref[r:r + 1, :])
        pre = acol * xb0 + bcol * xb1 + ccol
        enc = jnp.where(pre >= 0, pre, a * pre)
        h2 = _dot00(w2b, _bf16r(enc)) + b2col
        h2b = _bf16r(h2)
        y_ref[r:r + 1, :] = jnp.sum(wgf * h2b, axis=0, keepdims=True)
    del y_ref


def _t_body(c0_ref, c1_ref, y_ref, t_ref, dinv_ref):
    deg = c0_ref[...] + c1_ref[...] + 1.0
    dinv = lax.rsqrt(deg)
    t_ref[...] = dinv * y_ref[...]
    dinv_ref[...] = dinv


def _combine_body(p_ref, a0_ref, a1_ref, t_ref, dinv_ref, out_ref):
    cst = p_ref[0, 0]
    out_ref[...] = (dinv_ref[...]
                    * (a0_ref[...] + a1_ref[...] + t_ref[...]) + cst)


def kernel(x, edge_index, W1, b1, gamma, beta, prelu_a, W2, b2, Wg, bg, Wf, bf):
    N = x.shape[0]
    E = edge_index.shape[1]
    np_ = ((N + 1023) // 1024) * 1024
    rows = np_ // 128
    slc = np_ // NS
    per_tile = E // (NC * NS)
    chunk = 2000
    rblk = 56
    while rows % rblk:
        rblk -= 8
    f32 = jnp.float32

    pad = np_ - N
    x0p = jnp.pad(x[:, 0], (0, pad)).reshape(rows, 128)
    x1p = jnp.pad(x[:, 1], (0, pad)).reshape(rows, 128)

    mesh = plsc.VectorSubcoreMesh(core_axis_name="c", subcore_axis_name="s")
    sc_params = pltpu.CompilerParams(use_tc_tiling_on_sc=False,
                                     needs_layout_passes=False)
    cnt = pl.kernel(
        functools.partial(_hist_body, np_, per_tile, chunk),
        out_type=jax.ShapeDtypeStruct((NC, NS, slc), f32),
        mesh=mesh,
        compiler_params=sc_params,
        scratch_types=[
            pltpu.VMEM_SHARED((np_,), f32),
            pltpu.VMEM((chunk,), jnp.int32),
            pltpu.VMEM((chunk,), jnp.int32),
            pltpu.VMEM((chunk,), f32),
            pltpu.VMEM((slc,), f32),
            pltpu.SemaphoreType.DMA,
            pltpu.SemaphoreType.DMA,
        ],
    )(edge_index)
    cnt_r = cnt.reshape(NC, rows, 128)

    mom = pl.pallas_call(
        _moments_body,
        out_shape=jax.ShapeDtypeStruct((8, 128), f32),
    )(x0p, x1p)
    sums = jnp.sum(mom, axis=1)
    n_f = jnp.float32(N)
    m0, m1 = sums[0] / n_f, sums[1] / n_f
    e00, e11, e01 = sums[2] / n_f, sums[3] / n_f, sums[4] / n_f
    v00 = e00 - m0 * m0
    v01 = e01 - m0 * m1
    v11 = e11 - m1 * m1

    w1b = _bf16r(W1)
    mu = m0 * w1b[0] + m1 * w1b[1] + b1
    var = v00 * w1b[0] ** 2 + 2.0 * v01 * w1b[0] * w1b[1] + v11 * w1b[1] ** 2
    g = gamma * lax.rsqrt(var + 1e-5)
    acol = (g * w1b[0]).reshape(32, 1)
    bcol = (g * w1b[1]).reshape(32, 1)
    ccol = (g * (b1 - mu) + beta).reshape(32, 1)
    wgf = jnp.dot(Wg, Wf)
    cst = jnp.dot(bg, Wf)[0] + bf[0]
    dparams = jnp.reshape(cst, (1, 1))

    full = pl.BlockSpec(memory_space=pltpu.VMEM)
    y_r = pl.pallas_call(
        functools.partial(_encoder_body, rblk),
        grid=(rows // rblk,),
        in_specs=[pl.BlockSpec((rblk, 128), lambda i: (i, 0)),
                  pl.BlockSpec((rblk, 128), lambda i: (i, 0)),
                  pl.BlockSpec(memory_space=pltpu.SMEM),
                  full, full, full, full, full, full],
        out_specs=pl.BlockSpec((rblk, 128), lambda i: (i, 0)),
        out_shape=jax.ShapeDtypeStruct((rows, 128), f32),
    )(x0p, x1p, prelu_a.reshape(1, 1), acol, bcol, ccol, _bf16r(W2),
      b2.reshape(32, 1), wgf)

    t_r, dinv_r = pl.pallas_call(
        _t_body,
        out_shape=[jax.ShapeDtypeStruct((rows, 128), f32),
                   jax.ShapeDtypeStruct((rows, 128), f32)],
    )(cnt_r[0], cnt_r[1], y_r)

    acc = pl.kernel(
        functools.partial(_gs_body, np_, per_tile, chunk),
        out_type=jax.ShapeDtypeStruct((NC, NS, slc), f32),
        mesh=mesh,
        compiler_params=sc_params,
        scratch_types=[
            pltpu.VMEM_SHARED((np_,), f32),
            pltpu.VMEM((np_,), f32),
            pltpu.VMEM((chunk,), jnp.int32),
            pltpu.VMEM((chunk,), jnp.int32),
            pltpu.VMEM((chunk,), f32),
            pltpu.VMEM((chunk,), jnp.int32),
            pltpu.VMEM((chunk,), jnp.int32),
            pltpu.VMEM((chunk,), f32),
            pltpu.VMEM((slc,), f32),
            pltpu.SemaphoreType.DMA,
            pltpu.SemaphoreType.DMA,
        ],
    )(edge_index, t_r.reshape(np_))
    acc_r = acc.reshape(NC, rows, 128)

    scores_r = pl.pallas_call(
        _combine_body,
        in_specs=[pl.BlockSpec(memory_space=pltpu.SMEM),
                  full, full, full, full],
        out_shape=jax.ShapeDtypeStruct((rows, 128), f32),
    )(dparams, acc_r[0], acc_r[1], t_r, dinv_r)
    return scores_r.reshape(np_)[:N]
